# baseline (speedup 1.0000x reference)
import functools

import jax
import jax.numpy as jnp
from jax import lax
from jax.experimental import pallas as pl
from jax.experimental.pallas import tpu as pltpu
from jax.experimental.pallas import tpu_sc as plsc

N = 10000
E = 320000
H = 16
D_FEAT = 128
OUT = 128

_G = 128
_NGROUPS = E // _G
_INFO = plsc.get_sparse_core_info()
_NC = _INFO.num_cores
_NS = _INFO.num_subcores
_NW = _NC * _NS
_GPW = _NGROUPS // _NW
_EXTRA = _NGROUPS - _GPW * _NW
_NPAD = 10240
_CPT = _NPAD // _NS


def _sc_segment_sum(src2d, zT, zeros_init):
  mesh = plsc.VectorSubcoreMesh(core_axis_name="c", subcore_axis_name="s")

  @functools.partial(
      pl.kernel,
      out_type=jax.ShapeDtypeStruct((_NC, H, _NPAD), jnp.float32),
      mesh=mesh,
      compiler_params=pltpu.CompilerParams(use_tc_tiling_on_sc=False),
      scratch_types=[
          pltpu.VMEM((_GPW, 2, _G), jnp.int32),
          pltpu.VMEM((1, 2, _G), jnp.int32),
          pltpu.VMEM((_G,), jnp.int32),
          pltpu.VMEM((_G,), jnp.int32),
          pltpu.VMEM((_G,), jnp.int32),
          pltpu.VMEM((_G,), jnp.int32),
          pltpu.VMEM((H, _G), jnp.float32),
          pltpu.VMEM((H, _G), jnp.float32),
          pltpu.VMEM((H, _G), jnp.float32),
          pltpu.VMEM((H, _G), jnp.float32),
          pltpu.VMEM_SHARED((H, _NPAD), jnp.float32),
          pltpu.SemaphoreType.DMA,
          pltpu.SemaphoreType.DMA,
          pltpu.SemaphoreType.DMA,
          pltpu.SemaphoreType.DMA,
          pltpu.SemaphoreType.DMA,
          pltpu.SemaphoreType.DMA,
          pltpu.SemaphoreType.DMA,
          pltpu.SemaphoreType.DMA,
      ],
  )
  def seg_sum(src_hbm, zt_hbm, zero_hbm, out_hbm,
              idx2d, exidx, idxb0, idxb1, idxb2, idxb3,
              zb0, zb1, zb2, zb3, acc,
              sl0, sl1, sl2, sl3, ss0, ss1, ss2, ss3):
    c = lax.axis_index("c")
    s = lax.axis_index("s")
    wid = s * _NC + c

    pltpu.sync_copy(zero_hbm.at[:, pl.ds(s * _CPT, _CPT)],
                    acc.at[:, pl.ds(s * _CPT, _CPT)])
    pltpu.sync_copy(src_hbm.at[pl.ds(wid * _GPW, _GPW)], idx2d)
    plsc.subcore_barrier()

    g0 = wid * _GPW

    def issue(g_rel, zb, sem):
      pltpu.async_copy(zt_hbm.at[g0 + g_rel], zb, sem)

    def wait_ld(zb, sem):
      pltpu.make_async_copy(zt_hbm.at[0], zb, sem).wait()

    def fire(zb, idxb, sem):
      for f in range(H):
        pltpu.async_copy(zb.at[f], acc.at[f].at[idxb], sem, add=True)

    def drain(zb, idxb, sem):
      for f in range(H):
        pltpu.make_async_copy(zb.at[f], acc.at[f].at[idxb], sem).wait()

    def sync_scatter(zb, idxb, sem):
      fire(zb, idxb, sem)
      drain(zb, idxb, sem)

    ring = ((zb0, idxb0, sl0, ss0), (zb1, idxb1, sl1, ss1),
            (zb2, idxb2, sl2, ss2), (zb3, idxb3, sl3, ss3))

    issue(0, zb0, sl0)
    issue(1, zb1, sl1)
    issue(2, zb2, sl2)

    def quad_body(p, carry):
      for u, (zb, idxb, sl, ss) in enumerate(ring):
        g_rel = 4 * p + u
        zb_n, idxb_n, sl_n, ss_n = ring[(u + 1) % 4]

        @pl.when(g_rel < _GPW)
        def _():
          for k in range(_G // 16):
            idxb[pl.ds(k * 16, 16)] = idx2d[g_rel, 0, pl.ds(k * 16, 16)]
          wait_ld(zb, sl)
          fire(zb, idxb, ss)

        @pl.when((g_rel >= 3) & (g_rel - 3 < _GPW))
        def _():
          drain(zb_n, idxb_n, ss_n)

        @pl.when((g_rel + 1 < _GPW) & (g_rel >= 2))
        def _():
          issue(g_rel + 1, zb_n, sl_n)
      return carry

    lax.fori_loop(0, (_GPW + 6) // 4, quad_body, 0)

    @pl.when(wid < _EXTRA)
    def _():
      gx = _NW * _GPW + wid
      pltpu.sync_copy(src_hbm.at[pl.ds(gx, 1)], exidx)
      for k in range(_G // 16):
        idxb0[pl.ds(k * 16, 16)] = exidx[0, 0, pl.ds(k * 16, 16)]
      pltpu.sync_copy(zt_hbm.at[gx], zb0)
      sync_scatter(zb0, idxb0, ss0)

    plsc.subcore_barrier()
    pltpu.sync_copy(acc.at[:, pl.ds(s * _CPT, _CPT)],
                    out_hbm.at[c, :, pl.ds(s * _CPT, _CPT)])

  return seg_sum(src2d, zT, zeros_init)


_ZCH = 32000


def _zt_body(w0bT_ref, attrT_ref, zt_ref):
  zc = jnp.dot(w0bT_ref[...], attrT_ref[...],
               preferred_element_type=jnp.float32)
  for g in range(_ZCH // _G):
    zt_ref[g] = zc[:, g * _G:(g + 1) * _G]


def _xw_body(x_ref, w0a_ref, b0_ref, t_ref):
  t_ref[...] = (jnp.dot(x_ref[...], w0a_ref[...],
                        preferred_element_type=jnp.float32) + b0_ref[...])


def _elu(t):
  return jnp.where(t > 0, t, jnp.exp(jnp.minimum(t, 0.0)) - 1.0)


def _tail_body(t_ref, p_ref, w2_ref, b2_ref, w3_ref, b3_ref, o_ref):
  esT = p_ref[0, :, pl.ds(0, N)] + p_ref[1, :, pl.ds(0, N)]
  h = _elu(t_ref[...] + jnp.transpose(esT))
  h = _elu(jnp.dot(h, w2_ref[...], preferred_element_type=jnp.float32)
           + b2_ref[...])
  o_ref[...] = (jnp.dot(h, w3_ref[...], preferred_element_type=jnp.float32)
                + b3_ref[...])


def kernel(x, edge_index, edge_attr, W0, b0, W2, b2, W3, b3):
  src2d = jnp.transpose(
      edge_index.astype(jnp.int32).reshape(2, _NGROUPS, _G), (1, 0, 2))
  zeros_init = jnp.zeros((H, _NPAD), jnp.float32)
  w0a = W0[:D_FEAT]
  w0bT = W0[D_FEAT:].T

  zT = pl.pallas_call(
      _zt_body,
      grid=(E // _ZCH,),
      in_specs=[pl.BlockSpec((H, H), lambda i: (0, 0)),
                pl.BlockSpec((H, _ZCH), lambda i: (0, i))],
      out_specs=pl.BlockSpec((_ZCH // _G, H, _G), lambda i: (i, 0, 0)),
      out_shape=jax.ShapeDtypeStruct((_NGROUPS, H, _G), jnp.float32),
  )(w0bT, edge_attr.T)

  partials = _sc_segment_sum(src2d, zT, zeros_init)

  t0 = pl.pallas_call(
      _xw_body,
      out_shape=jax.ShapeDtypeStruct((N, H), jnp.float32),
  )(x, w0a, b0.reshape(1, H))

  out = pl.pallas_call(
      _tail_body,
      out_shape=jax.ShapeDtypeStruct((N, OUT), jnp.float32),
  )(t0, partials, W2, b2.reshape(1, H), W3, b3.reshape(1, OUT))
  return out


# speedup vs baseline: 1.0682x; 1.0682x over previous
import functools

import jax
import jax.numpy as jnp
from jax import lax
from jax.experimental import pallas as pl
from jax.experimental.pallas import tpu as pltpu
from jax.experimental.pallas import tpu_sc as plsc

N = 10000
E = 320000
H = 16
D_FEAT = 128
OUT = 128

_G = 128
_NGROUPS = E // _G
_INFO = plsc.get_sparse_core_info()
_NC = _INFO.num_cores
_NS = _INFO.num_subcores
_NW = _NC * _NS
_GPW = _NGROUPS // _NW
_EXTRA = _NGROUPS - _GPW * _NW
_NPAD = 10240
_CPT = _NPAD // _NS


def _sc_segment_sum(src2d, zT, zeros_init):
  mesh = plsc.VectorSubcoreMesh(core_axis_name="c", subcore_axis_name="s")

  @functools.partial(
      pl.kernel,
      out_type=jax.ShapeDtypeStruct((_NC, H, _NPAD), jnp.float32),
      mesh=mesh,
      compiler_params=pltpu.CompilerParams(use_tc_tiling_on_sc=False),
      scratch_types=[
          pltpu.VMEM((_GPW, 2, _G), jnp.int32),
          pltpu.VMEM((1, 2, _G), jnp.int32),
          pltpu.VMEM((_G,), jnp.int32),
          pltpu.VMEM((_G,), jnp.int32),
          pltpu.VMEM((_G,), jnp.int32),
          pltpu.VMEM((_G,), jnp.int32),
          pltpu.VMEM((H, _G), jnp.float32),
          pltpu.VMEM((H, _G), jnp.float32),
          pltpu.VMEM((H, _G), jnp.float32),
          pltpu.VMEM((H, _G), jnp.float32),
          pltpu.VMEM_SHARED((H, _NPAD), jnp.float32),
          pltpu.SemaphoreType.DMA,
          pltpu.SemaphoreType.DMA,
          pltpu.SemaphoreType.DMA,
          pltpu.SemaphoreType.DMA,
          pltpu.SemaphoreType.DMA,
          pltpu.SemaphoreType.DMA,
          pltpu.SemaphoreType.DMA,
          pltpu.SemaphoreType.DMA,
      ],
  )
  def seg_sum(src_hbm, zt_hbm, zero_hbm, out_hbm,
              idx2d, exidx, idxb0, idxb1, idxb2, idxb3,
              zb0, zb1, zb2, zb3, acc,
              sl0, sl1, sl2, sl3, ss0, ss1, ss2, ss3):
    c = lax.axis_index("c")
    s = lax.axis_index("s")
    wid = s * _NC + c

    pltpu.sync_copy(zero_hbm.at[:, pl.ds(s * _CPT, _CPT)],
                    acc.at[:, pl.ds(s * _CPT, _CPT)])
    pltpu.sync_copy(src_hbm.at[pl.ds(wid * _GPW, _GPW)], idx2d)
    plsc.subcore_barrier()

    g0 = wid * _GPW

    def issue(g_rel, zb, sem):
      pltpu.async_copy(zt_hbm.at[g0 + g_rel], zb, sem)

    def wait_ld(zb, sem):
      pltpu.make_async_copy(zt_hbm.at[0], zb, sem).wait()

    def fire(zb, idxb, sem):
      for f in range(H):
        pltpu.async_copy(zb.at[f], acc.at[f].at[idxb], sem, add=True)

    def drain(zb, idxb, sem):
      for f in range(H):
        pltpu.make_async_copy(zb.at[f], acc.at[f].at[idxb], sem).wait()

    def sync_scatter(zb, idxb, sem):
      fire(zb, idxb, sem)
      drain(zb, idxb, sem)

    ring = ((zb0, idxb0, sl0, ss0), (zb1, idxb1, sl1, ss1),
            (zb2, idxb2, sl2, ss2), (zb3, idxb3, sl3, ss3))

    issue(0, zb0, sl0)
    issue(1, zb1, sl1)

    def quad_body(p, carry):
      for u, (zb, idxb, sl, ss) in enumerate(ring):
        g_rel = 4 * p + u
        zb_n, idxb_n, sl_n, ss_n = ring[(u + 2) % 4]

        @pl.when(g_rel < _GPW)
        def _():
          for k in range(_G // 16):
            idxb[pl.ds(k * 16, 16)] = idx2d[g_rel, 0, pl.ds(k * 16, 16)]
          wait_ld(zb, sl)
          fire(zb, idxb, ss)

        @pl.when((g_rel >= 2) & (g_rel - 2 < _GPW))
        def _():
          drain(zb_n, idxb_n, ss_n)

        @pl.when(g_rel + 2 < _GPW)
        def _():
          issue(g_rel + 2, zb_n, sl_n)
      return carry

    lax.fori_loop(0, (_GPW + 5) // 4, quad_body, 0)

    @pl.when(wid < _EXTRA)
    def _():
      gx = _NW * _GPW + wid
      pltpu.sync_copy(src_hbm.at[pl.ds(gx, 1)], exidx)
      for k in range(_G // 16):
        idxb0[pl.ds(k * 16, 16)] = exidx[0, 0, pl.ds(k * 16, 16)]
      pltpu.sync_copy(zt_hbm.at[gx], zb0)
      sync_scatter(zb0, idxb0, ss0)

    plsc.subcore_barrier()
    pltpu.sync_copy(acc.at[:, pl.ds(s * _CPT, _CPT)],
                    out_hbm.at[c, :, pl.ds(s * _CPT, _CPT)])

  return seg_sum(src2d, zT, zeros_init)


_ZCH = 32000


def _zt_body(w0bT_ref, attrT_ref, zt_ref):
  zc = jnp.dot(w0bT_ref[...], attrT_ref[...],
               preferred_element_type=jnp.float32)
  for g in range(_ZCH // _G):
    zt_ref[g] = zc[:, g * _G:(g + 1) * _G]


def _xw_body(x_ref, w0a_ref, b0_ref, t_ref):
  t_ref[...] = (jnp.dot(x_ref[...], w0a_ref[...],
                        preferred_element_type=jnp.float32) + b0_ref[...])


def _elu(t):
  return jnp.where(t > 0, t, jnp.exp(jnp.minimum(t, 0.0)) - 1.0)


def _tail_body(t_ref, p_ref, w2_ref, b2_ref, w3_ref, b3_ref, o_ref):
  esT = p_ref[0, :, pl.ds(0, N)] + p_ref[1, :, pl.ds(0, N)]
  h = _elu(t_ref[...] + jnp.transpose(esT))
  h = _elu(jnp.dot(h, w2_ref[...], preferred_element_type=jnp.float32)
           + b2_ref[...])
  o_ref[...] = (jnp.dot(h, w3_ref[...], preferred_element_type=jnp.float32)
                + b3_ref[...])


def kernel(x, edge_index, edge_attr, W0, b0, W2, b2, W3, b3):
  src2d = jnp.transpose(
      edge_index.astype(jnp.int32).reshape(2, _NGROUPS, _G), (1, 0, 2))
  zeros_init = jnp.zeros((H, _NPAD), jnp.float32)
  w0a = W0[:D_FEAT]
  w0bT = W0[D_FEAT:].T

  zT = pl.pallas_call(
      _zt_body,
      grid=(E // _ZCH,),
      in_specs=[pl.BlockSpec((H, H), lambda i: (0, 0)),
                pl.BlockSpec((H, _ZCH), lambda i: (0, i))],
      out_specs=pl.BlockSpec((_ZCH // _G, H, _G), lambda i: (i, 0, 0)),
      out_shape=jax.ShapeDtypeStruct((_NGROUPS, H, _G), jnp.float32),
  )(w0bT, edge_attr.T)

  partials = _sc_segment_sum(src2d, zT, zeros_init)

  t0 = pl.pallas_call(
      _xw_body,
      out_shape=jax.ShapeDtypeStruct((N, H), jnp.float32),
  )(x, w0a, b0.reshape(1, H))

  out = pl.pallas_call(
      _tail_body,
      out_shape=jax.ShapeDtypeStruct((N, OUT), jnp.float32),
  )(t0, partials, W2, b2.reshape(1, H), W3, b3.reshape(1, OUT))
  return out
